# baseline (speedup 1.0000x reference)
import functools
import jax
import jax.numpy as jnp
from jax import lax
from jax.experimental import pallas as pl
from jax.experimental.pallas import tpu as pltpu
from jax.experimental.pallas import tpu_sc as plsc

N_USER = 100000
N_ITEM = 50000
EMBED = 128

TC_GRID = 4
TC_ROWS = 25000

NW = 32
I_SLAB = 1568
CHUNK = 504


def _tc_copy(u_in, u_out):
    u_out[...] = u_in[...]


def _chunks(total):
    sizes = [CHUNK] * (total // CHUNK)
    if total % CHUNK:
        sizes.append(total % CHUNK)
    offs = [CHUNK * i for i in range(len(sizes))]
    return list(zip(offs, sizes))


def _sc_copy(i_in, i_out, buf_a, buf_b, rs_a, rs_b, ws_a, ws_b):
    wid = lax.axis_index("s") * 2 + lax.axis_index("c")
    i_base = jnp.minimum(wid * I_SLAB, N_ITEM - I_SLAB)

    work = _chunks(I_SLAB)
    bufs = [buf_a, buf_b]
    rsems = [rs_a, rs_b]
    wsems = [ws_a, ws_b]

    writes = [None] * len(work)
    for idx, (off, sz) in enumerate(work):
        b = idx % 2
        if idx >= 2:
            writes[idx - 2].wait()
        rd = pltpu.make_async_copy(
            i_in.at[pl.ds(i_base + off, sz)], bufs[b].at[pl.ds(0, sz)],
            rsems[b])
        rd.start()
        rd.wait()
        wr = pltpu.make_async_copy(
            bufs[b].at[pl.ds(0, sz)], i_out.at[pl.ds(i_base + off, sz)],
            wsems[b])
        wr.start()
        writes[idx] = wr
    writes[-2].wait()
    writes[-1].wait()


def kernel(embed_user, embed_item):
    mesh = plsc.VectorSubcoreMesh(core_axis_name="c", subcore_axis_name="s")
    sc_k = functools.partial(
        pl.kernel,
        mesh=mesh,
        out_type=jax.ShapeDtypeStruct((N_ITEM, EMBED), jnp.float32),
        scratch_types=[
            pltpu.VMEM((CHUNK, EMBED), jnp.float32),
            pltpu.VMEM((CHUNK, EMBED), jnp.float32),
            pltpu.SemaphoreType.DMA,
            pltpu.SemaphoreType.DMA,
            pltpu.SemaphoreType.DMA,
            pltpu.SemaphoreType.DMA,
        ],
    )(_sc_copy)
    item_out = sc_k(embed_item)

    user_out = embed_user + jnp.float32(0.0)
    return (user_out, item_out)


# speedup vs baseline: 1.4893x; 1.4893x over previous
import jax
import jax.numpy as jnp
from jax.experimental import pallas as pl

N_GRID = 5
USER_ROWS = 20000
ITEM_ROWS = 10000
EMBED = 128


def _copy_kernel(user_in, item_in, user_out, item_out):
    user_out[...] = user_in[...]
    item_out[...] = item_in[...]


def kernel(embed_user, embed_item):
    return tuple(pl.pallas_call(
        _copy_kernel,
        grid=(N_GRID,),
        in_specs=[
            pl.BlockSpec((USER_ROWS, EMBED), lambda i: (i, 0)),
            pl.BlockSpec((ITEM_ROWS, EMBED), lambda i: (i, 0)),
        ],
        out_specs=[
            pl.BlockSpec((USER_ROWS, EMBED), lambda i: (i, 0)),
            pl.BlockSpec((ITEM_ROWS, EMBED), lambda i: (i, 0)),
        ],
        out_shape=[
            jax.ShapeDtypeStruct(embed_user.shape, embed_user.dtype),
            jax.ShapeDtypeStruct(embed_item.shape, embed_item.dtype),
        ],
    )(embed_user, embed_item))
